# baseline (speedup 1.0000x reference)
import jax
import jax.numpy as jnp
from jax.experimental import pallas as pl

_SPLIT = 2


def _moe_body(x_ref, idx_ref, w_ref, wg_ref, wu_ref, dn_ref, out_ref):
    e = pl.program_id(0)
    c = pl.program_id(1)
    x = x_ref[...]
    gate = jax.lax.dot_general(
        x, wg_ref[0], (((1,), (1,)), ((), ())),
        preferred_element_type=jnp.float32)
    up = jax.lax.dot_general(
        x, wu_ref[0], (((1,), (1,)), ((), ())),
        preferred_element_type=jnp.float32)
    h = gate * jax.nn.sigmoid(gate) * up
    oe = jax.lax.dot_general(
        h, dn_ref[0], (((1,), (1,)), ((), ())),
        preferred_element_type=jnp.float32)
    cw = jnp.sum(
        jnp.where(idx_ref[...] == e, w_ref[...], 0.0), axis=1)
    contrib = oe * cw[:, None]

    @pl.when((e == 0) & (c == 0))
    def _init():
        out_ref[...] = contrib

    @pl.when((e != 0) | (c != 0))
    def _acc():
        out_ref[...] += contrib


def kernel(hidden_states, top_k_index, top_k_weights, gate_up_proj, down_proj):
    T, H = hidden_states.shape
    E, I2, _ = gate_up_proj.shape
    I = down_proj.shape[-1]
    S = _SPLIT
    IC = I // S

    return pl.pallas_call(
        _moe_body,
        grid=(E, S),
        in_specs=[
            pl.BlockSpec((T, H), lambda e, c: (0, 0)),
            pl.BlockSpec(top_k_index.shape, lambda e, c: (0, 0)),
            pl.BlockSpec(top_k_weights.shape, lambda e, c: (0, 0)),
            pl.BlockSpec((1, IC, H), lambda e, c: (e, c, 0)),
            pl.BlockSpec((1, IC, H), lambda e, c: (e, S + c, 0)),
            pl.BlockSpec((1, H, IC), lambda e, c: (e, 0, c)),
        ],
        out_specs=pl.BlockSpec((T, H), lambda e, c: (0, 0)),
        out_shape=jax.ShapeDtypeStruct((T, H), jnp.float32),
    )(hidden_states, top_k_index, top_k_weights,
      gate_up_proj, gate_up_proj, down_proj)


# speedup vs baseline: 1.1165x; 1.1165x over previous
import jax
import jax.numpy as jnp
from jax import lax
from jax.experimental import pallas as pl
from jax.experimental.pallas import tpu as pltpu
from jax.experimental.pallas import tpu_sc as plsc


def _combine_sc_body(idx_hbm, w_hbm, out_hbm, idx_v, w_v, fi_v, z_v, comb_sh):
    cid = lax.axis_index("c")
    sid = lax.axis_index("s")
    tk = idx_v.shape[0]
    et = z_v.shape[0]
    t_sz = 64

    @pl.when((cid == 0) & (sid == 0))
    def _():
        pltpu.sync_copy(idx_hbm, idx_v)
        pltpu.sync_copy(w_hbm, w_v)
        zeros = jnp.zeros((16,), jnp.float32)
        for j in range(et // 16):
            z_v[pl.ds(j * 16, 16)] = zeros
        lanes = lax.iota(jnp.int32, 16)
        for j in range(tk // 16):
            t = lanes + (j % (t_sz // 16)) * 16
            fi_v[pl.ds(j * 16, 16)] = idx_v[pl.ds(j * 16, 16)] * t_sz + t
        pltpu.sync_copy(z_v, comb_sh)
        pltpu.sync_copy(w_v, comb_sh.at[fi_v], add=True)
        pltpu.sync_copy(comb_sh, out_hbm)


def _routing_combine(top_k_index, top_k_weights, E, T):
    K = top_k_index.shape[1]
    idx_flat = top_k_index.T.reshape(T * K)
    w_flat = top_k_weights.T.reshape(T * K)
    kern = pl.kernel(
        _combine_sc_body,
        mesh=plsc.VectorSubcoreMesh(core_axis_name="c", subcore_axis_name="s"),
        out_type=jax.ShapeDtypeStruct((E * T,), jnp.float32),
        scratch_types=[
            pltpu.VMEM((T * K,), jnp.int32),
            pltpu.VMEM((T * K,), jnp.float32),
            pltpu.VMEM((T * K,), jnp.int32),
            pltpu.VMEM((E * T,), jnp.float32),
            pltpu.VMEM_SHARED((E * T,), jnp.float32),
        ],
    )
    return kern(idx_flat, w_flat).reshape(E, 1, T)


def _moe_body(x_ref, comb_ref, wg_ref, wu_ref, dn_ref, out_ref):
    e = pl.program_id(0)
    x = x_ref[...]
    gate = jax.lax.dot_general(
        x, wg_ref[0], (((1,), (1,)), ((), ())),
        preferred_element_type=jnp.float32)
    up = jax.lax.dot_general(
        x, wu_ref[0], (((1,), (1,)), ((), ())),
        preferred_element_type=jnp.float32)
    h = gate * jax.nn.sigmoid(gate) * up
    oe = jax.lax.dot_general(
        h, dn_ref[0], (((1,), (1,)), ((), ())),
        preferred_element_type=jnp.float32)
    cw = comb_ref[0, 0, :][:, None]
    contrib = oe * cw

    @pl.when(e == 0)
    def _init():
        out_ref[...] = contrib

    @pl.when(e != 0)
    def _acc():
        out_ref[...] += contrib


def kernel(hidden_states, top_k_index, top_k_weights, gate_up_proj, down_proj):
    T, H = hidden_states.shape
    E, I2, _ = gate_up_proj.shape
    I = down_proj.shape[-1]

    combine = _routing_combine(top_k_index, top_k_weights, E, T)

    return pl.pallas_call(
        _moe_body,
        grid=(E,),
        in_specs=[
            pl.BlockSpec((T, H), lambda e: (0, 0)),
            pl.BlockSpec((1, 1, T), lambda e: (e, 0, 0)),
            pl.BlockSpec((1, I, H), lambda e: (e, 0, 0)),
            pl.BlockSpec((1, I, H), lambda e: (e, 1, 0)),
            pl.BlockSpec((1, H, I), lambda e: (e, 0, 0)),
        ],
        out_specs=pl.BlockSpec((T, H), lambda e: (0, 0)),
        out_shape=jax.ShapeDtypeStruct((T, H), jnp.float32),
    )(hidden_states, combine, gate_up_proj, gate_up_proj, down_proj)


# speedup vs baseline: 1.4326x; 1.2831x over previous
import jax
import jax.numpy as jnp
from jax.experimental import pallas as pl


_EPB = 2


def _moe_body(x_ref, idx_ref, w_ref, wg_ref, wu_ref, dn_ref, out_ref):
    g = pl.program_id(0)
    x = x_ref[...]
    acc = None
    for ee in range(_EPB):
        e = g * _EPB + ee
        gate = jax.lax.dot_general(
            x, wg_ref[ee, 0], (((1,), (1,)), ((), ())),
            preferred_element_type=jnp.float32)
        up = jax.lax.dot_general(
            x, wu_ref[ee, 0], (((1,), (1,)), ((), ())),
            preferred_element_type=jnp.float32)
        h = gate * jax.nn.sigmoid(gate) * up
        oe = jax.lax.dot_general(
            h, dn_ref[ee], (((1,), (1,)), ((), ())),
            preferred_element_type=jnp.float32)
        cw = jnp.sum(
            jnp.where(idx_ref[...] == e, w_ref[...], 0.0), axis=1)
        contrib = oe * cw[:, None]
        acc = contrib if acc is None else acc + contrib

    @pl.when(g == 0)
    def _init():
        out_ref[...] = acc

    @pl.when(g != 0)
    def _acc():
        out_ref[...] += acc


def kernel(hidden_states, top_k_index, top_k_weights, gate_up_proj, down_proj):
    T, H = hidden_states.shape
    E, I2, _ = gate_up_proj.shape
    I = down_proj.shape[-1]
    B = _EPB

    gu4 = gate_up_proj.reshape(E, 2, I, H)

    return pl.pallas_call(
        _moe_body,
        grid=(E // B,),
        in_specs=[
            pl.BlockSpec((T, H), lambda g: (0, 0)),
            pl.BlockSpec(top_k_index.shape, lambda g: (0, 0)),
            pl.BlockSpec(top_k_weights.shape, lambda g: (0, 0)),
            pl.BlockSpec((B, 1, I, H), lambda g: (g, 0, 0, 0)),
            pl.BlockSpec((B, 1, I, H), lambda g: (g, 1, 0, 0)),
            pl.BlockSpec((B, H, I), lambda g: (g, 0, 0)),
        ],
        out_specs=pl.BlockSpec((T, H), lambda g: (0, 0)),
        out_shape=jax.ShapeDtypeStruct((T, H), jnp.float32),
    )(hidden_states, top_k_index, top_k_weights, gu4, gu4, down_proj)
